# baseline (speedup 1.0000x reference)
import functools

import jax
import jax.numpy as jnp
import numpy as np
from jax import lax
from jax.experimental import pallas as pl
from jax.experimental.pallas import tpu as pltpu
from jax.experimental.pallas import tpu_sc as plsc

N = 10000
E = 160000
DF = 128
H = 64
F = 32
NI = 3
G = 64
CUT = 8.0

NC = 2
NS = 16
NW = NC * NS
CHUNK = 128
E_PAD = 163840
CPW = E_PAD // (NW * CHUNK)
CPW0 = 40
CPW1 = 40
NCHUNKS = E_PAD // CHUNK
META_ROWS = 1344
NPAD = 10240
ROWS_PER_TILE = NPAD // NS


def _embed_body(x_ref, w_ref, b_ref, o_ref):
    o_ref[...] = (
        jnp.dot(x_ref[...], w_ref[...], preferred_element_type=jnp.float32)
        + b_ref[...]
    )


def _embed(x, emb_W, emb_b):
    blk = 1000
    return pl.pallas_call(
        _embed_body,
        grid=(N // blk,),
        in_specs=[
            pl.BlockSpec((blk, DF), lambda i: (i, 0)),
            pl.BlockSpec((DF, H), lambda i: (0, 0)),
            pl.BlockSpec((1, H), lambda i: (0, 0)),
        ],
        out_specs=pl.BlockSpec((blk, H), lambda i: (i, 0)),
        out_shape=jax.ShapeDtypeStruct((N, H), jnp.float32),
    )(x, emb_W, emb_b.reshape(1, H))


def _edge_scalar_body(ew_ref, w1_ref, b1_ref, w2_ref, b2_ref, o_ref):
    ew = ew_ref[0]
    scaled = ew * (2.0 / CUT) - 1.0
    cut = jnp.where(ew <= CUT, 0.5 * (jnp.cos(ew * (np.pi / CUT)) + 1.0), 0.0)
    for i in range(NI):
        f = jnp.tanh(w1_ref[i] * scaled + b1_ref[i])
        f2 = jnp.dot(w2_ref[i], f, preferred_element_type=jnp.float32) + b2_ref[i]
        o_ref[i, 0] = jnp.sum(f2 * cut, axis=0, keepdims=True)


EB = 4096


def _edge_scalars(ew_pad, w1T, b1T, w2T, b2T):
    rows = E_PAD // EB
    return pl.pallas_call(
        _edge_scalar_body,
        grid=(rows,),
        in_specs=[
            pl.BlockSpec((1, 1, EB), lambda i: (i, 0, 0)),
            pl.BlockSpec((NI, F, 1), lambda i: (0, 0, 0)),
            pl.BlockSpec((NI, F, 1), lambda i: (0, 0, 0)),
            pl.BlockSpec((NI, F, F), lambda i: (0, 0, 0)),
            pl.BlockSpec((NI, F, 1), lambda i: (0, 0, 0)),
        ],
        out_specs=pl.BlockSpec((NI, 1, 1, EB), lambda i: (0, i, 0, 0)),
        out_shape=jax.ShapeDtypeStruct((NI, rows, 1, EB), jnp.float32),
    )(ew_pad, w1T, b1T, w2T, b2T)


def _softplus(x):
    return jnp.maximum(x, 0.0) + jnp.log1p(jnp.exp(-jnp.abs(x)))


def _interact_body(agg_ref, h_ref, w1_ref, b1_ref, w2_ref, b2_ref,
                   g_ref, be_ref, mu_ref, var_ref, o_ref):
    agg = agg_ref[...]
    t = _softplus(
        jnp.dot(agg, w1_ref[...], preferred_element_type=jnp.float32) + b1_ref[...]
    )
    y = jnp.dot(t, w2_ref[...], preferred_element_type=jnp.float32) + b2_ref[...]
    y = g_ref[...] * (y - mu_ref[...]) / jnp.sqrt(var_ref[...] + 1e-3) + be_ref[...]
    o_ref[...] = h_ref[...] + y


def _interact(agg, h, iW1, ib1, iW2, ib2, g, be, mu, var):
    blk = 1000
    vec = pl.BlockSpec((1, H), lambda i: (0, 0))
    return pl.pallas_call(
        _interact_body,
        grid=(N // blk,),
        in_specs=[
            pl.BlockSpec((blk, H), lambda i: (i, 0)),
            pl.BlockSpec((blk, H), lambda i: (i, 0)),
            pl.BlockSpec((H, H), lambda i: (0, 0)),
            vec,
            pl.BlockSpec((H, H), lambda i: (0, 0)),
            vec, vec, vec, vec, vec,
        ],
        out_specs=pl.BlockSpec((blk, H), lambda i: (i, 0)),
        out_shape=jax.ShapeDtypeStruct((N, H), jnp.float32),
    )(agg, h, iW1, ib1.reshape(1, H), iW2, ib2.reshape(1, H),
      g.reshape(1, H), be.reshape(1, H), mu.reshape(1, H), var.reshape(1, H))


def _pool_body(batch_ref, h_ref, o_ref):
    i = pl.program_id(0)
    bi = batch_ref[0, 0, :]
    gids = lax.broadcasted_iota(jnp.int32, (G, bi.shape[0]), 0)
    P = (gids == bi[None, :]).astype(jnp.float32)
    part = lax.dot_general(P, h_ref[...], (((1,), (0,)), ((), ())),
                           precision=lax.Precision.HIGHEST,
                           preferred_element_type=jnp.float32)

    @pl.when(i == 0)
    def _():
        o_ref[...] = jnp.zeros_like(o_ref)

    o_ref[...] += part


def _pool(h, batch_idx):
    blk = 1000
    nb = N // blk
    batch3 = batch_idx.reshape(nb, 1, blk)
    return pl.pallas_call(
        _pool_body,
        grid=(nb,),
        in_specs=[
            pl.BlockSpec((1, 1, blk), lambda i: (i, 0, 0)),
            pl.BlockSpec((blk, H), lambda i: (i, 0)),
        ],
        out_specs=pl.BlockSpec((G, H), lambda i: (0, 0)),
        out_shape=jax.ShapeDtypeStruct((G, H), jnp.float32),
    )(batch3, h)


def _outmlp_body(p_ref, w1_ref, b1_ref, w2_ref, b2_ref, w3_ref, b3_ref, o_ref):
    o = _softplus(
        jnp.dot(p_ref[...], w1_ref[...], preferred_element_type=jnp.float32)
        + b1_ref[...]
    )
    o = _softplus(
        jnp.dot(o, w2_ref[...], preferred_element_type=jnp.float32) + b2_ref[...]
    )
    o_ref[...] = (
        jnp.dot(o, w3_ref[...], preferred_element_type=jnp.float32) + b3_ref[...]
    )


def _outmlp(pooled, oW1, ob1, oW2, ob2, oW3, ob3):
    return pl.pallas_call(
        _outmlp_body,
        out_shape=jax.ShapeDtypeStruct((G, 1), jnp.float32),
    )(pooled, oW1, ob1.reshape(1, H // 2), oW2, ob2.reshape(1, H // 2),
      oW3, ob3.reshape(1, 1))


RPT0 = 320
RPT1 = 320
STMAX = 8192 + 16
SCAN_E = 8192
NSCAN = E_PAD // SCAN_E
EPT = E_PAD // NS


def _sc_edge_body(h_hbm, pk_hbm, s_hbm, out_hbm,
                  pk_sh, s_sh, pkb, svb, st_lr, st_col, st_s,
                  acc, g0, g1, gsem0, gsem1):
    cid = lax.axis_index("c")
    sid = lax.axis_index("s")
    own = jnp.where(cid == 0, RPT0, RPT1)
    base_r = jnp.where(cid == 0, sid * RPT0, NS * RPT0 + sid * RPT1)

    pltpu.sync_copy(pk_hbm.at[pl.ds(sid * EPT, EPT)], pk_sh.at[pl.ds(sid * EPT, EPT)])
    pltpu.sync_copy(s_hbm.at[pl.ds(sid * EPT, EPT)], s_sh.at[pl.ds(sid * EPT, EPT)])

    zf = jnp.zeros((16,), jnp.float32)
    zi = jnp.zeros((16,), jnp.int32)

    def zacc(r, c):
        for q in range(H // 16):
            acc[r, pl.ds(q * 16, 16)] = zf
        return c

    lax.fori_loop(0, own, zacc, 0)

    def zstag(r, c):
        sl = pl.ds(r * 16, 16)
        st_lr[sl] = zi
        st_col[sl] = zi
        st_s[sl] = zf
        return c

    lax.fori_loop(0, STMAX // 16, zstag, 0)

    plsc.subcore_barrier()

    def scan_blk(b, cnt):
        pltpu.sync_copy(pk_sh.at[pl.ds(b * SCAN_E, SCAN_E)], pkb)
        pltpu.sync_copy(s_sh.at[pl.ds(b * SCAN_E, SCAN_E)], svb)

        def scan_vreg(v, c2):
            sl = pl.ds(v * 16, 16)
            pk = pkb[sl]
            row = lax.shift_right_logical(pk, 16)
            colv = lax.bitwise_and(pk, 0xFFFF)
            m = (row >= base_r) & (row < base_r + own)
            n16 = plsc.all_reduce_population_count(m)
            lr = jnp.clip(row - base_r, 0, own - 1)
            cv = jnp.clip(colv, 0, N - 1)
            c2c = jnp.clip(c2, 0, STMAX - 16)
            plsc.store_compressed(st_lr.at[pl.ds(c2c, 16)], lr, mask=m)
            plsc.store_compressed(st_col.at[pl.ds(c2c, 16)], cv, mask=m)
            plsc.store_compressed(st_s.at[pl.ds(c2c, 16)], svb[sl], mask=m)
            return jnp.clip(c2 + n16[0], 0, STMAX - 16)

        return lax.fori_loop(0, SCAN_E // 16, scan_vreg, cnt)

    count = lax.fori_loop(0, NSCAN, scan_blk, 0)
    nwin = (count + CHUNK - 1) // CHUNK

    gbuf = (g0, g1)
    gsem = (gsem0, gsem1)

    def start_gather(w, p):
        pltpu.async_copy(h_hbm.at[st_col.at[pl.ds(w * CHUNK, CHUNK)]],
                         gbuf[p], gsem[p])

    def wait_gather(w, p):
        pltpu.make_async_copy(h_hbm.at[st_col.at[pl.ds(w * CHUNK, CHUNK)]],
                              gbuf[p], gsem[p]).wait()

    @pl.when(nwin > 0)
    def _():
        start_gather(0, 0)

    @pl.when(nwin > 1)
    def _():
        start_gather(1, 1)

    def pair_body(k, carry):
        for p in range(2):
            w = 2 * k + p

            @pl.when(w < nwin)
            def _():
                wait_gather(w, p)

                def add_grp(g, c2):
                    base16 = w * CHUNK + g * 16
                    sv16 = st_s[pl.ds(base16, 16)]
                    lr16 = st_lr[pl.ds(base16, 16)]
                    for t in range(16):
                        sval = sv16[t]
                        lr = lr16[t]
                        for q in range(H // 16):
                            sl = pl.ds(q * 16, 16)
                            acc[lr, sl] = acc[lr, sl] + gbuf[p][g * 16 + t, sl] * sval
                    return c2

                lax.fori_loop(0, CHUNK // 16, add_grp, 0, unroll=2)

                @pl.when(w + 2 < nwin)
                def _():
                    start_gather(w + 2, p)
        return carry

    lax.fori_loop(0, (nwin + 1) // 2, pair_body, 0)

    @pl.when(cid == 0)
    def _():
        pltpu.sync_copy(acc.at[pl.ds(0, RPT0)], out_hbm.at[pl.ds(base_r, RPT0)])

    @pl.when(cid == 1)
    def _():
        pltpu.sync_copy(acc.at[pl.ds(0, RPT1)], out_hbm.at[pl.ds(base_r, RPT1)])


def _sc_edge(h, packed, s):
    mesh = plsc.VectorSubcoreMesh(core_axis_name="c", subcore_axis_name="s")
    kern = pl.kernel(
        _sc_edge_body,
        out_type=jax.ShapeDtypeStruct((NPAD, H), jnp.float32),
        mesh=mesh,
        compiler_params=pltpu.CompilerParams(use_tc_tiling_on_sc=False,
                                             needs_layout_passes=False),
        scratch_types=[
            pltpu.VMEM_SHARED((E_PAD,), jnp.int32),
            pltpu.VMEM_SHARED((E_PAD,), jnp.float32),
            pltpu.VMEM((SCAN_E,), jnp.int32),
            pltpu.VMEM((SCAN_E,), jnp.float32),
            pltpu.VMEM((STMAX,), jnp.int32),
            pltpu.VMEM((STMAX,), jnp.int32),
            pltpu.VMEM((STMAX,), jnp.float32),
            pltpu.VMEM((RPT0, H), jnp.float32),
            pltpu.VMEM((CHUNK, H), jnp.float32),
            pltpu.VMEM((CHUNK, H), jnp.float32),
            pltpu.SemaphoreType.DMA,
            pltpu.SemaphoreType.DMA,
        ],
    )
    return kern(h, packed, s)


def kernel(x, edge_index, edge_weight, edge_attr, batch_idx, emb_W, emb_b,
           fW1, fb1, fW2, fb2, iW1, ib1, iW2, ib2, bn_gamma, bn_beta,
           bn_mean, bn_var, oW1, ob1, oW2, ob2, oW3, ob3):
    w1T = jnp.transpose(fW1, (0, 2, 1))
    b1T = fb1[:, :, None]
    w2T = jnp.transpose(fW2, (0, 2, 1))
    b2T = fb2[:, :, None]

    pad = E_PAD - E
    col_p = jnp.concatenate([edge_index[1], jnp.zeros((pad,), jnp.int32)])
    row_p = jnp.concatenate([edge_index[0], jnp.zeros((pad,), jnp.int32)])
    ew_p = jnp.concatenate([edge_weight,
                            jnp.full((pad,), 2.0 * CUT, jnp.float32)])
    packed = row_p * 65536 + col_p

    S = _edge_scalars(ew_p.reshape(E_PAD // EB, 1, EB), w1T, b1T, w2T, b2T)

    h = _embed(x, emb_W, emb_b)

    for i in range(NI):
        agg = _sc_edge(h, packed, S[i].reshape(E_PAD))[:N]
        h = _interact(agg, h, iW1[i], ib1[i], iW2[i], ib2[i],
                      bn_gamma[i], bn_beta[i], bn_mean[i], bn_var[i])

    pooled = _pool(h, batch_idx)
    o = _outmlp(pooled, oW1, ob1, oW2, ob2, oW3, ob3)
    return jnp.squeeze(o, -1)


# speedup vs baseline: 1.0843x; 1.0843x over previous
import functools

import jax
import jax.numpy as jnp
import numpy as np
from jax import lax
from jax.experimental import pallas as pl
from jax.experimental.pallas import tpu as pltpu
from jax.experimental.pallas import tpu_sc as plsc

N = 10000
E = 160000
DF = 128
H = 64
F = 32
NI = 3
G = 64
CUT = 8.0

NC = 2
NS = 16
NW = NC * NS
CHUNK = 128
E_PAD = 163840
CPW = E_PAD // (NW * CHUNK)
CPW0 = 40
CPW1 = 40
NCHUNKS = E_PAD // CHUNK
META_ROWS = 1344
NPAD = 10240
ROWS_PER_TILE = NPAD // NS


def _embed_body(x_ref, w_ref, b_ref, o_ref):
    o_ref[...] = (
        jnp.dot(x_ref[...], w_ref[...], preferred_element_type=jnp.float32)
        + b_ref[...]
    )


def _embed(x, emb_W, emb_b):
    blk = 1000
    return pl.pallas_call(
        _embed_body,
        grid=(N // blk,),
        in_specs=[
            pl.BlockSpec((blk, DF), lambda i: (i, 0)),
            pl.BlockSpec((DF, H), lambda i: (0, 0)),
            pl.BlockSpec((1, H), lambda i: (0, 0)),
        ],
        out_specs=pl.BlockSpec((blk, H), lambda i: (i, 0)),
        out_shape=jax.ShapeDtypeStruct((N, H), jnp.float32),
    )(x, emb_W, emb_b.reshape(1, H))


def _edge_scalar_body(ew_ref, w1_ref, b1_ref, w2_ref, b2_ref, o_ref):
    ew = ew_ref[0]
    scaled = ew * (2.0 / CUT) - 1.0
    cut = jnp.where(ew <= CUT, 0.5 * (jnp.cos(ew * (np.pi / CUT)) + 1.0), 0.0)
    for i in range(NI):
        f = jnp.tanh(w1_ref[i] * scaled + b1_ref[i])
        f2 = jnp.dot(w2_ref[i], f, preferred_element_type=jnp.float32) + b2_ref[i]
        o_ref[i, 0] = jnp.sum(f2 * cut, axis=0, keepdims=True)


EB = 4096


def _edge_scalars(ew_pad, w1T, b1T, w2T, b2T):
    rows = E_PAD // EB
    return pl.pallas_call(
        _edge_scalar_body,
        grid=(rows,),
        in_specs=[
            pl.BlockSpec((1, 1, EB), lambda i: (i, 0, 0)),
            pl.BlockSpec((NI, F, 1), lambda i: (0, 0, 0)),
            pl.BlockSpec((NI, F, 1), lambda i: (0, 0, 0)),
            pl.BlockSpec((NI, F, F), lambda i: (0, 0, 0)),
            pl.BlockSpec((NI, F, 1), lambda i: (0, 0, 0)),
        ],
        out_specs=pl.BlockSpec((NI, 1, 1, EB), lambda i: (0, i, 0, 0)),
        out_shape=jax.ShapeDtypeStruct((NI, rows, 1, EB), jnp.float32),
    )(ew_pad, w1T, b1T, w2T, b2T)


def _softplus(x):
    return jnp.maximum(x, 0.0) + jnp.log1p(jnp.exp(-jnp.abs(x)))


def _interact_body(agg_ref, h_ref, w1_ref, b1_ref, w2_ref, b2_ref,
                   g_ref, be_ref, mu_ref, var_ref, o_ref):
    agg = agg_ref[...]
    t = _softplus(
        jnp.dot(agg, w1_ref[...], preferred_element_type=jnp.float32) + b1_ref[...]
    )
    y = jnp.dot(t, w2_ref[...], preferred_element_type=jnp.float32) + b2_ref[...]
    y = g_ref[...] * (y - mu_ref[...]) / jnp.sqrt(var_ref[...] + 1e-3) + be_ref[...]
    o_ref[...] = h_ref[...] + y


def _interact(agg, h, iW1, ib1, iW2, ib2, g, be, mu, var):
    blk = 1000
    vec = pl.BlockSpec((1, H), lambda i: (0, 0))
    return pl.pallas_call(
        _interact_body,
        grid=(N // blk,),
        in_specs=[
            pl.BlockSpec((blk, H), lambda i: (i, 0)),
            pl.BlockSpec((blk, H), lambda i: (i, 0)),
            pl.BlockSpec((H, H), lambda i: (0, 0)),
            vec,
            pl.BlockSpec((H, H), lambda i: (0, 0)),
            vec, vec, vec, vec, vec,
        ],
        out_specs=pl.BlockSpec((blk, H), lambda i: (i, 0)),
        out_shape=jax.ShapeDtypeStruct((N, H), jnp.float32),
    )(agg, h, iW1, ib1.reshape(1, H), iW2, ib2.reshape(1, H),
      g.reshape(1, H), be.reshape(1, H), mu.reshape(1, H), var.reshape(1, H))


def _pool_body(batch_ref, h_ref, o_ref):
    i = pl.program_id(0)
    bi = batch_ref[0, 0, :]
    gids = lax.broadcasted_iota(jnp.int32, (G, bi.shape[0]), 0)
    P = (gids == bi[None, :]).astype(jnp.float32)
    part = lax.dot_general(P, h_ref[...], (((1,), (0,)), ((), ())),
                           precision=lax.Precision.HIGHEST,
                           preferred_element_type=jnp.float32)

    @pl.when(i == 0)
    def _():
        o_ref[...] = jnp.zeros_like(o_ref)

    o_ref[...] += part


def _pool(h, batch_idx):
    blk = 1000
    nb = N // blk
    batch3 = batch_idx.reshape(nb, 1, blk)
    return pl.pallas_call(
        _pool_body,
        grid=(nb,),
        in_specs=[
            pl.BlockSpec((1, 1, blk), lambda i: (i, 0, 0)),
            pl.BlockSpec((blk, H), lambda i: (i, 0)),
        ],
        out_specs=pl.BlockSpec((G, H), lambda i: (0, 0)),
        out_shape=jax.ShapeDtypeStruct((G, H), jnp.float32),
    )(batch3, h)


def _outmlp_body(p_ref, w1_ref, b1_ref, w2_ref, b2_ref, w3_ref, b3_ref, o_ref):
    o = _softplus(
        jnp.dot(p_ref[...], w1_ref[...], preferred_element_type=jnp.float32)
        + b1_ref[...]
    )
    o = _softplus(
        jnp.dot(o, w2_ref[...], preferred_element_type=jnp.float32) + b2_ref[...]
    )
    o_ref[...] = (
        jnp.dot(o, w3_ref[...], preferred_element_type=jnp.float32) + b3_ref[...]
    )


def _outmlp(pooled, oW1, ob1, oW2, ob2, oW3, ob3):
    return pl.pallas_call(
        _outmlp_body,
        out_shape=jax.ShapeDtypeStruct((G, 1), jnp.float32),
    )(pooled, oW1, ob1.reshape(1, H // 2), oW2, ob2.reshape(1, H // 2),
      oW3, ob3.reshape(1, 1))


RPT0 = 480
RPT1 = 160
STMAX = 8192 + 16
SCAN_E = 8192
NSCAN = E_PAD // SCAN_E
EPT = E_PAD // NS


def _sc_edge_body(h_hbm, pk_hbm, s_hbm, out_hbm,
                  pk_sh, s_sh, pkb, svb, st_lr, st_col, st_s,
                  acc, g0, g1, gsem0, gsem1):
    cid = lax.axis_index("c")
    sid = lax.axis_index("s")
    own = jnp.where(cid == 0, RPT0, RPT1)
    base_r = jnp.where(cid == 0, sid * RPT0, NS * RPT0 + sid * RPT1)

    pltpu.sync_copy(pk_hbm.at[pl.ds(sid * EPT, EPT)], pk_sh.at[pl.ds(sid * EPT, EPT)])
    pltpu.sync_copy(s_hbm.at[pl.ds(sid * EPT, EPT)], s_sh.at[pl.ds(sid * EPT, EPT)])

    zf = jnp.zeros((16,), jnp.float32)
    zi = jnp.zeros((16,), jnp.int32)

    def zacc(r, c):
        for q in range(H // 16):
            acc[r, pl.ds(q * 16, 16)] = zf
        return c

    lax.fori_loop(0, own, zacc, 0)

    def zstag(r, c):
        sl = pl.ds(r * 16, 16)
        st_lr[sl] = zi
        st_col[sl] = zi
        st_s[sl] = zf
        return c

    lax.fori_loop(0, STMAX // 16, zstag, 0)

    plsc.subcore_barrier()

    def scan_blk(b, cnt):
        pltpu.sync_copy(pk_sh.at[pl.ds(b * SCAN_E, SCAN_E)], pkb)
        pltpu.sync_copy(s_sh.at[pl.ds(b * SCAN_E, SCAN_E)], svb)

        def scan_vreg(v, c2):
            sl = pl.ds(v * 16, 16)
            pk = pkb[sl]
            row = lax.shift_right_logical(pk, 16)
            colv = lax.bitwise_and(pk, 0xFFFF)
            m = (row >= base_r) & (row < base_r + own)
            n16 = plsc.all_reduce_population_count(m)
            lr = jnp.clip(row - base_r, 0, own - 1)
            cv = jnp.clip(colv, 0, N - 1)
            c2c = jnp.clip(c2, 0, STMAX - 16)
            plsc.store_compressed(st_lr.at[pl.ds(c2c, 16)], lr, mask=m)
            plsc.store_compressed(st_col.at[pl.ds(c2c, 16)], cv, mask=m)
            plsc.store_compressed(st_s.at[pl.ds(c2c, 16)], svb[sl], mask=m)
            return jnp.clip(c2 + n16[0], 0, STMAX - 16)

        return lax.fori_loop(0, SCAN_E // 16, scan_vreg, cnt)

    count = lax.fori_loop(0, NSCAN, scan_blk, 0)
    nwin = (count + CHUNK - 1) // CHUNK

    gbuf = (g0, g1)
    gsem = (gsem0, gsem1)

    def start_gather(w, p):
        pltpu.async_copy(h_hbm.at[st_col.at[pl.ds(w * CHUNK, CHUNK)]],
                         gbuf[p], gsem[p])

    def wait_gather(w, p):
        pltpu.make_async_copy(h_hbm.at[st_col.at[pl.ds(w * CHUNK, CHUNK)]],
                              gbuf[p], gsem[p]).wait()

    @pl.when(nwin > 0)
    def _():
        start_gather(0, 0)

    @pl.when(nwin > 1)
    def _():
        start_gather(1, 1)

    def pair_body(k, carry):
        for p in range(2):
            w = 2 * k + p

            @pl.when(w < nwin)
            def _():
                wait_gather(w, p)

                def add_grp(g, c2):
                    base16 = w * CHUNK + g * 16
                    sv16 = st_s[pl.ds(base16, 16)]
                    lr16 = st_lr[pl.ds(base16, 16)]
                    for t in range(16):
                        sval = sv16[t]
                        lr = lr16[t]
                        for q in range(H // 16):
                            sl = pl.ds(q * 16, 16)
                            acc[lr, sl] = acc[lr, sl] + gbuf[p][g * 16 + t, sl] * sval
                    return c2

                lax.fori_loop(0, CHUNK // 16, add_grp, 0, unroll=2)

                @pl.when(w + 2 < nwin)
                def _():
                    start_gather(w + 2, p)
        return carry

    lax.fori_loop(0, (nwin + 1) // 2, pair_body, 0)

    @pl.when(cid == 0)
    def _():
        pltpu.sync_copy(acc.at[pl.ds(0, RPT0)], out_hbm.at[pl.ds(base_r, RPT0)])

    @pl.when(cid == 1)
    def _():
        pltpu.sync_copy(acc.at[pl.ds(0, RPT1)], out_hbm.at[pl.ds(base_r, RPT1)])


def _sc_edge(h, packed, s):
    mesh = plsc.VectorSubcoreMesh(core_axis_name="c", subcore_axis_name="s")
    kern = pl.kernel(
        _sc_edge_body,
        out_type=jax.ShapeDtypeStruct((NPAD, H), jnp.float32),
        mesh=mesh,
        compiler_params=pltpu.CompilerParams(use_tc_tiling_on_sc=False,
                                             needs_layout_passes=False),
        scratch_types=[
            pltpu.VMEM_SHARED((E_PAD,), jnp.int32),
            pltpu.VMEM_SHARED((E_PAD,), jnp.float32),
            pltpu.VMEM((SCAN_E,), jnp.int32),
            pltpu.VMEM((SCAN_E,), jnp.float32),
            pltpu.VMEM((STMAX,), jnp.int32),
            pltpu.VMEM((STMAX,), jnp.int32),
            pltpu.VMEM((STMAX,), jnp.float32),
            pltpu.VMEM((RPT0, H), jnp.float32),
            pltpu.VMEM((CHUNK, H), jnp.float32),
            pltpu.VMEM((CHUNK, H), jnp.float32),
            pltpu.SemaphoreType.DMA,
            pltpu.SemaphoreType.DMA,
        ],
    )
    return kern(h, packed, s)


def kernel(x, edge_index, edge_weight, edge_attr, batch_idx, emb_W, emb_b,
           fW1, fb1, fW2, fb2, iW1, ib1, iW2, ib2, bn_gamma, bn_beta,
           bn_mean, bn_var, oW1, ob1, oW2, ob2, oW3, ob3):
    w1T = jnp.transpose(fW1, (0, 2, 1))
    b1T = fb1[:, :, None]
    w2T = jnp.transpose(fW2, (0, 2, 1))
    b2T = fb2[:, :, None]

    pad = E_PAD - E
    col_p = jnp.concatenate([edge_index[1], jnp.zeros((pad,), jnp.int32)])
    row_p = jnp.concatenate([edge_index[0], jnp.zeros((pad,), jnp.int32)])
    ew_p = jnp.concatenate([edge_weight,
                            jnp.full((pad,), 2.0 * CUT, jnp.float32)])
    packed = row_p * 65536 + col_p

    S = _edge_scalars(ew_p.reshape(E_PAD // EB, 1, EB), w1T, b1T, w2T, b2T)

    h = _embed(x, emb_W, emb_b)

    for i in range(NI):
        agg = _sc_edge(h, packed, S[i].reshape(E_PAD))[:N]
        h = _interact(agg, h, iW1[i], ib1[i], iW2[i], ib2[i],
                      bn_gamma[i], bn_beta[i], bn_mean[i], bn_var[i])

    pooled = _pool(h, batch_idx)
    o = _outmlp(pooled, oW1, ob1, oW2, ob2, oW3, ob3)
    return jnp.squeeze(o, -1)
